# Initial kernel scaffold; baseline (speedup 1.0000x reference)
import jax
import jax.numpy as jnp
from jax.experimental import pallas as pl


def kernel(inputs, centroids, W):
    raise NotImplementedError("write your pallas kernel here")


# speedup vs baseline: 2.5672x; 2.5672x over previous
import jax
import jax.numpy as jnp
from jax import lax
from jax.experimental import pallas as pl
from jax.experimental.pallas import tpu as pltpu

_NCENT = 512
_NCB = 32
_SUB = 64
_BT = 256
_DIN = _NCB * _SUB


def _fused_body(x_ref, cent_ref, w_ref, prod_ref, negmse_ref, codes_ref, outs_ref):
    x = x_ref[...]
    negs = []
    codes = []
    for c in range(_NCB):
        xc = x[:, c * _SUB:(c + 1) * _SUB]
        cc = cent_ref[c]
        resp = lax.dot_general(xc, cc, (((1,), (1,)), ((), ())))
        m = jnp.max(resp, axis=-1, keepdims=True)
        iota = lax.broadcasted_iota(jnp.int32, resp.shape, 1)
        code = jnp.min(jnp.where(resp == m, iota, _NCENT), axis=-1, keepdims=True)
        onehot = (iota == code).astype(jnp.float32)
        gathered = lax.dot_general(onehot, cc, (((1,), (0,)), ((), ())))
        outs_ref[:, c * _SUB:(c + 1) * _SUB] = gathered
        negs.append(-m)
        codes.append(code)
    negmse_ref[...] = jnp.concatenate(negs, axis=1)
    codes_ref[...] = jnp.concatenate(codes, axis=1)
    prod_ref[...] = lax.dot_general(outs_ref[...], w_ref[...],
                                    (((1,), (0,)), ((), ())))


def kernel(inputs, centroids, W):
    B = inputs.shape[0]
    out_dim = W.shape[1]
    x2 = inputs.reshape(B, _DIN)
    grid = (B // _BT,)
    prod, negmse, codes = pl.pallas_call(
        _fused_body,
        grid=grid,
        in_specs=[
            pl.BlockSpec((_BT, _DIN), lambda i: (i, 0)),
            pl.BlockSpec((_NCB, _NCENT, _SUB), lambda i: (0, 0, 0)),
            pl.BlockSpec((_DIN, out_dim), lambda i: (0, 0)),
        ],
        out_specs=(
            pl.BlockSpec((_BT, out_dim), lambda i: (i, 0)),
            pl.BlockSpec((_BT, _NCB), lambda i: (i, 0)),
            pl.BlockSpec((_BT, _NCB), lambda i: (i, 0)),
        ),
        out_shape=(
            jax.ShapeDtypeStruct((B, out_dim), jnp.float32),
            jax.ShapeDtypeStruct((B, _NCB), jnp.float32),
            jax.ShapeDtypeStruct((B, _NCB), jnp.int32),
        ),
        scratch_shapes=[pltpu.VMEM((_BT, _DIN), jnp.float32)],
        compiler_params=pltpu.CompilerParams(
            dimension_semantics=("arbitrary",),
        ),
    )(x2, centroids, W)
    return (prod, negmse, codes)
